# baseline (speedup 1.0000x reference)
import functools

import jax
import jax.numpy as jnp
from jax import lax
from jax.experimental import pallas as pl
from jax.experimental.pallas import tpu as pltpu
from jax.experimental.pallas import tpu_sc as plsc

E = 8
K = 2
BM_R = 512
M = 256
NC, NS = 2, 16
NW = NC * NS
GC = 16
GS = 8
CC = 16


def _router_body(x_ref, rw_ref, rb_ref, sel_ref, wts_ref, xc_ref):
    xc_ref[...] = x_ref[...]
    logits = lax.dot_general(
        x_ref[...], rw_ref[...], (((1,), (1,)), ((), ())),
        preferred_element_type=jnp.float32) + rb_ref[...]
    iota = lax.broadcasted_iota(jnp.int32, logits.shape, 1)
    m1 = jnp.max(logits, axis=1, keepdims=True)
    idx1 = jnp.min(jnp.where(logits == m1, iota, E), axis=1, keepdims=True)
    masked = jnp.where(iota == idx1, -jnp.inf, logits)
    m2 = jnp.max(masked, axis=1, keepdims=True)
    idx2 = jnp.min(jnp.where(masked == m2, iota, E), axis=1, keepdims=True)
    p = jnp.exp(m2 - m1)
    w1 = 1.0 / (1.0 + p)
    col = lax.broadcasted_iota(jnp.int32, (logits.shape[0], K), 1)
    sel_ref[...] = jnp.where(col == 0, idx1, idx2)
    wts_ref[...] = jnp.where(col == 0, w1, 1.0 - w1)


def _route(x, router_W, rb2, interpret=False):
    B, D = x.shape
    return pl.pallas_call(
        _router_body,
        grid=(B // BM_R,),
        in_specs=[
            pl.BlockSpec((BM_R, D), lambda t: (t, 0)),
            pl.BlockSpec((E, D), lambda t: (0, 0)),
            pl.BlockSpec((1, E), lambda t: (0, 0)),
        ],
        out_specs=[
            pl.BlockSpec((BM_R, K), lambda t: (t, 0)),
            pl.BlockSpec((BM_R, K), lambda t: (t, 0)),
            pl.BlockSpec((BM_R, D), lambda t: (t, 0)),
        ],
        out_shape=[
            jax.ShapeDtypeStruct((B, K), jnp.int32),
            jax.ShapeDtypeStruct((B, K), jnp.float32),
            jax.ShapeDtypeStruct((B, D), jnp.float32),
        ],
        interpret=interpret,
    )(x, router_W, rb2)


def _metadata(sel, wts):
    B = sel.shape[0]
    P = B * K
    G = P // M + E
    Pmax = G * M
    flat_e = sel.reshape(P)
    j = jnp.arange(P, dtype=jnp.int32)
    rev = jnp.zeros_like(j)
    for i in range(8):
        rev = rev | (((j >> (i + 1)) & 1) << (7 - i))
    key = (flat_e << 21) | (rev << 13) | j
    sort_idx = jnp.argsort(key)
    sorted_e = jnp.take(flat_e, sort_idx)
    sizes = jnp.bincount(flat_e, length=E)
    starts = jnp.concatenate([jnp.zeros((1,), jnp.int32),
                              jnp.cumsum(sizes)[:-1].astype(jnp.int32)])
    nblk = (sizes + (M - 1)) // M
    cnb = jnp.cumsum(nblk).astype(jnp.int32)
    blk_first = jnp.concatenate([jnp.zeros((1,), jnp.int32), cnb[:-1]])
    eg = jnp.clip(jnp.searchsorted(cnb, jnp.arange(G, dtype=jnp.int32),
                                   side="right"), 0, E - 1).astype(jnp.int32)
    q = jnp.arange(P, dtype=jnp.int32) - jnp.take(starts, sorted_e)
    rowpos = (jnp.take(blk_first, sorted_e) + q // M) * M + q % M
    row_tok = jnp.zeros((Pmax,), jnp.int32).at[rowpos].set(sort_idx // K)
    row_w = jnp.zeros((Pmax,), jnp.float32).at[rowpos].set(
        jnp.take(wts.reshape(P), sort_idx))
    inv = jnp.zeros((P,), jnp.int32).at[sort_idx].set(rowpos)
    return eg, row_tok, row_w, inv[0::2], inv[1::2]


def _sc_gather(x, row_tok, Pmax, D):
    nch_total = Pmax // GC
    cpw = nch_total // NW
    mesh = plsc.VectorSubcoreMesh(core_axis_name="c", subcore_axis_name="s")

    @functools.partial(
        pl.kernel, mesh=mesh,
        out_type=jax.ShapeDtypeStruct((Pmax, D), jnp.float32),
        scratch_types=[
            pltpu.VMEM((Pmax,), jnp.int32),
            pltpu.VMEM((2, GC, D), jnp.float32),
            pltpu.SemaphoreType.DMA,
            pltpu.SemaphoreType.DMA,
            pltpu.SemaphoreType.DMA,
            pltpu.SemaphoreType.DMA,
        ],
    )
    def gather(x_hbm, tok_hbm, xg_hbm, idx_v, buf_v, si0, si1, so0, so1):
        wid = lax.axis_index("s") * NC + lax.axis_index("c")
        pltpu.sync_copy(tok_hbm, idx_v)
        sin = [si0, si1]
        sout = [so0, so1]
        din = [[], []]
        dout = [None, None]

        def chunk_off(c):
            return pl.multiple_of((wid + c * NW) * GC, 8)

        def issue(c):
            o = chunk_off(c)
            b = c % 2
            din[b] = []
            for s in range(0, GC, GS):
                din[b].append(pltpu.async_copy(
                    x_hbm.at[idx_v.at[pl.ds(o + s, GS)]],
                    buf_v.at[b, pl.ds(s, GS)], sin[b]))

        def drain(c):
            o = chunk_off(c)
            b = c % 2
            for d in din[b]:
                d.wait()
            dout[b] = pltpu.async_copy(
                buf_v.at[b], xg_hbm.at[pl.ds(o, GC)], sout[b])

        for c in range(cpw):
            b = c % 2
            if dout[b] is not None:
                dout[b].wait()
            issue(c)
            if c >= 1:
                drain(c - 1)
        drain(cpw - 1)
        for b in range(2):
            if dout[b] is not None:
                dout[b].wait()

    return gather(x, row_tok)


def _gmm_body(eg_ref, xg_ref, ew_ref, eb_ref, w_ref, y_ref):
    y = lax.dot_general(
        xg_ref[...], ew_ref[0], (((1,), (1,)), ((), ())),
        preferred_element_type=jnp.float32)
    y_ref[...] = (y + eb_ref[0]) * w_ref[...]


def _gmm(eg, xg, expert_weight, eb3, row_w, interpret=False):
    Pmax, D = xg.shape
    O = expert_weight.shape[1]
    G = Pmax // M
    grid_spec = pltpu.PrefetchScalarGridSpec(
        num_scalar_prefetch=1,
        grid=(G,),
        in_specs=[
            pl.BlockSpec((M, D), lambda g, eg_s: (g, 0)),
            pl.BlockSpec((1, O, D), lambda g, eg_s: (eg_s[g], 0, 0)),
            pl.BlockSpec((1, 1, O), lambda g, eg_s: (eg_s[g], 0, 0)),
            pl.BlockSpec((M, 1), lambda g, eg_s: (g, 0)),
        ],
        out_specs=pl.BlockSpec((M, O), lambda g, eg_s: (g, 0)),
    )
    return pl.pallas_call(
        _gmm_body,
        grid_spec=grid_spec,
        out_shape=jax.ShapeDtypeStruct((Pmax, O), jnp.float32),
        interpret=interpret,
    )(eg, xg, expert_weight, eb3, row_w.reshape(Pmax, 1))


def _sc_combine(y, inv0, inv1, B, O):
    tpw = B // NW
    cpw = tpw // CC
    mesh = plsc.VectorSubcoreMesh(core_axis_name="c", subcore_axis_name="s")

    @functools.partial(
        pl.kernel, mesh=mesh,
        out_type=jax.ShapeDtypeStruct((B, O), jnp.float32),
        scratch_types=[
            pltpu.VMEM((tpw,), jnp.int32),
            pltpu.VMEM((tpw,), jnp.int32),
            pltpu.VMEM((CC, O), jnp.float32),
            pltpu.VMEM((CC, O), jnp.float32),
            pltpu.SemaphoreType.DMA,
            pltpu.SemaphoreType.DMA,
        ],
    )
    def combine(y_hbm, i0_hbm, i1_hbm, out_hbm, i0_v, i1_v, bufa, bufb,
                sema, semb):
        wid = lax.axis_index("s") * NC + lax.axis_index("c")
        base = pl.multiple_of(wid * tpw, 8)
        pltpu.sync_copy(i0_hbm.at[pl.ds(base, tpw)], i0_v)
        pltpu.sync_copy(i1_hbm.at[pl.ds(base, tpw)], i1_v)
        for c in range(cpw):
            ds = []
            for s in range(0, CC, GS):
                ds.append(pltpu.async_copy(
                    y_hbm.at[i0_v.at[pl.ds(c * CC + s, GS)]],
                    bufa.at[pl.ds(s, GS)], sema))
                ds.append(pltpu.async_copy(
                    y_hbm.at[i1_v.at[pl.ds(c * CC + s, GS)]],
                    bufb.at[pl.ds(s, GS)], semb))
            for d in ds:
                d.wait()

            def add_row(i, _):
                def add_chunk(j, _):
                    j64 = pl.multiple_of(j * 64, 64)
                    for u in range(4):
                        sl = pl.ds(j64 + u * 16, 16)
                        bufa[i, sl] = bufa[i, sl] + bufb[i, sl]
                    return 0

                return lax.fori_loop(0, O // 64, add_chunk, 0)

            lax.fori_loop(0, CC, add_row, 0)
            pltpu.sync_copy(bufa, out_hbm.at[pl.ds(base + c * CC, CC)])

    return combine(y, inv0, inv1)


@functools.partial(jax.jit, static_argnames=("interpret",))
def kernel(x, router_W, router_b, expert_weight, expert_bias, interpret=False):
    B, D = x.shape
    O = expert_weight.shape[1]
    P = B * K
    G = P // M + E
    Pmax = G * M
    rb2 = router_b.reshape(1, E)
    eb3 = expert_bias.reshape(E, 1, O)
    sel, wts, xc = _route(x, router_W, rb2, interpret=interpret)
    eg, row_tok, row_w, inv0, inv1 = _metadata(sel, wts)
    if interpret:
        xg = jnp.take(x, row_tok, axis=0)
        y = _gmm(eg, xg, expert_weight, eb3, row_w, interpret=True)
        return jnp.take(y, inv0, axis=0) + jnp.take(y, inv1, axis=0)
    xg = _sc_gather(xc, row_tok, Pmax, D)
    y = _gmm(eg, xg, expert_weight, eb3, row_w)
    return _sc_combine(y, inv0, inv1, B, O)


# speedup vs baseline: 1.4070x; 1.4070x over previous
import functools

import jax
import jax.numpy as jnp
from jax.experimental import pallas as pl
from jax.experimental.pallas import tpu as pltpu

NUM_EXPERTS = 8
TOP_K = 2
BM = 512


def _topk2_weights(logits):
    E = logits.shape[1]
    iota = jax.lax.broadcasted_iota(jnp.int32, logits.shape, 1)
    m1 = jnp.max(logits, axis=1, keepdims=True)
    eq1 = logits == m1
    idx1 = jnp.min(jnp.where(eq1, iota, E), axis=1, keepdims=True)
    first = iota == idx1
    masked = jnp.where(first, -jnp.inf, logits)
    m2 = jnp.max(masked, axis=1, keepdims=True)
    eq2 = masked == m2
    idx2 = jnp.min(jnp.where(eq2, iota, E), axis=1, keepdims=True)
    second = iota == idx2
    p = jnp.exp(m2 - m1)
    denom = 1.0 + p
    w = (first.astype(jnp.float32) + second.astype(jnp.float32) * p) / denom
    return w


def _moe_body(x_ref, rw_ref, rb_ref, ew_ref, eb_ref, out_ref, w_scr):
    e = pl.program_id(1)

    @pl.when(e == 0)
    def _():
        logits = jax.lax.dot_general(
            x_ref[...], rw_ref[...],
            (((1,), (1,)), ((), ())),
            preferred_element_type=jnp.float32,
        ) + rb_ref[...]
        w_scr[...] = _topk2_weights(logits)
        out_ref[...] = jnp.zeros_like(out_ref)

    w_all = w_scr[...]
    onehot = jax.lax.broadcasted_iota(jnp.int32, w_all.shape, 1) == e
    w_e = jnp.sum(jnp.where(onehot, w_all, 0.0), axis=1, keepdims=True)
    y = jax.lax.dot_general(
        x_ref[...], ew_ref[0],
        (((1,), (1,)), ((), ())),
        preferred_element_type=jnp.float32,
    ) + eb_ref[0]
    out_ref[...] += y * w_e


@functools.partial(jax.jit, static_argnames=("interpret",))
def kernel(x, router_W, router_b, expert_weight, expert_bias, interpret=False):
    B, D = x.shape
    E, O, _ = expert_weight.shape
    rb2 = router_b.reshape(1, E)
    eb3 = expert_bias.reshape(E, 1, O)
    grid = (B // BM, E)
    out = pl.pallas_call(
        _moe_body,
        grid=grid,
        in_specs=[
            pl.BlockSpec((BM, D), lambda t, e: (t, 0)),
            pl.BlockSpec((E, D), lambda t, e: (0, 0)),
            pl.BlockSpec((1, E), lambda t, e: (0, 0)),
            pl.BlockSpec((1, O, D), lambda t, e: (e, 0, 0)),
            pl.BlockSpec((1, 1, O), lambda t, e: (e, 0, 0)),
        ],
        out_specs=pl.BlockSpec((BM, O), lambda t, e: (t, 0)),
        out_shape=jax.ShapeDtypeStruct((B, O), jnp.float32),
        scratch_shapes=[pltpu.VMEM((BM, NUM_EXPERTS), jnp.float32)],
        interpret=interpret,
    )(x, router_W, rb2, expert_weight, eb3)
    return out


# speedup vs baseline: 1.6345x; 1.1617x over previous
import functools

import jax
import jax.numpy as jnp
from jax.experimental import pallas as pl
from jax.experimental.pallas import tpu as pltpu

NUM_EXPERTS = 8
TOP_K = 2
BM = 1024
NS = 2


def _topk2_weights(logits):
    E = logits.shape[1]
    iota = jax.lax.broadcasted_iota(jnp.int32, logits.shape, 1)
    m1 = jnp.max(logits, axis=1, keepdims=True)
    eq1 = logits == m1
    idx1 = jnp.min(jnp.where(eq1, iota, E), axis=1, keepdims=True)
    first = iota == idx1
    masked = jnp.where(first, -jnp.inf, logits)
    m2 = jnp.max(masked, axis=1, keepdims=True)
    eq2 = masked == m2
    idx2 = jnp.min(jnp.where(eq2, iota, E), axis=1, keepdims=True)
    second = iota == idx2
    p = jnp.exp(m2 - m1)
    denom = 1.0 + p
    w = (first.astype(jnp.float32) + second.astype(jnp.float32) * p) / denom
    return w


def _moe_body(x_ref, rw_ref, rb_ref, ew_ref, eb_ref, out_ref, w_scr):
    n = pl.program_id(1)
    e = pl.program_id(2)

    @pl.when((n == 0) & (e == 0))
    def _():
        logits = jax.lax.dot_general(
            x_ref[...], rw_ref[...],
            (((1,), (1,)), ((), ())),
            preferred_element_type=jnp.float32,
        ) + rb_ref[...]
        w_scr[...] = _topk2_weights(logits)

    @pl.when(e == 0)
    def _():
        out_ref[...] = jnp.zeros_like(out_ref)

    w_all = w_scr[...]
    onehot = jax.lax.broadcasted_iota(jnp.int32, w_all.shape, 1) == e
    w_e = jnp.sum(jnp.where(onehot, w_all, 0.0), axis=1, keepdims=True)
    y = jax.lax.dot_general(
        x_ref[...], ew_ref[0],
        (((1,), (1,)), ((), ())),
        preferred_element_type=jnp.float32,
    ) + eb_ref[0]
    out_ref[...] += y * w_e


@functools.partial(jax.jit, static_argnames=("interpret",))
def kernel(x, router_W, router_b, expert_weight, expert_bias, interpret=False):
    B, D = x.shape
    E, O, _ = expert_weight.shape
    ON = O // NS
    rb2 = router_b.reshape(1, E)
    eb3 = expert_bias.reshape(E, 1, O)
    grid = (B // BM, NS, E)
    out = pl.pallas_call(
        _moe_body,
        grid=grid,
        in_specs=[
            pl.BlockSpec((BM, D), lambda t, n, e: (t, 0)),
            pl.BlockSpec((E, D), lambda t, n, e: (0, 0)),
            pl.BlockSpec((1, E), lambda t, n, e: (0, 0)),
            pl.BlockSpec((1, ON, D), lambda t, n, e: (e, n, 0)),
            pl.BlockSpec((1, 1, ON), lambda t, n, e: (e, 0, n)),
        ],
        out_specs=pl.BlockSpec((BM, ON), lambda t, n, e: (t, n)),
        out_shape=jax.ShapeDtypeStruct((B, O), jnp.float32),
        scratch_shapes=[pltpu.VMEM((BM, NUM_EXPERTS), jnp.float32)],
        interpret=interpret,
    )(x, router_W, rb2, expert_weight, eb3)
    return out


# speedup vs baseline: 1.6425x; 1.0049x over previous
import functools

import jax
import jax.numpy as jnp
from jax.experimental import pallas as pl
from jax.experimental.pallas import tpu as pltpu

NUM_EXPERTS = 8
TOP_K = 2
BM = 2048
NS = 4


def _topk2_weights(logits):
    E = logits.shape[1]
    iota = jax.lax.broadcasted_iota(jnp.int32, logits.shape, 1)
    m1 = jnp.max(logits, axis=1, keepdims=True)
    eq1 = logits == m1
    idx1 = jnp.min(jnp.where(eq1, iota, E), axis=1, keepdims=True)
    first = iota == idx1
    masked = jnp.where(first, -jnp.inf, logits)
    m2 = jnp.max(masked, axis=1, keepdims=True)
    eq2 = masked == m2
    idx2 = jnp.min(jnp.where(eq2, iota, E), axis=1, keepdims=True)
    second = iota == idx2
    p = jnp.exp(m2 - m1)
    denom = 1.0 + p
    w = (first.astype(jnp.float32) + second.astype(jnp.float32) * p) / denom
    return w


def _moe_body(x_ref, rw_ref, rb_ref, ew_ref, eb_ref, out_ref, w_scr):
    n = pl.program_id(1)
    e = pl.program_id(2)

    @pl.when((n == 0) & (e == 0))
    def _():
        logits = jax.lax.dot_general(
            x_ref[...], rw_ref[...],
            (((1,), (1,)), ((), ())),
            preferred_element_type=jnp.float32,
        ) + rb_ref[...]
        w_scr[...] = _topk2_weights(logits)

    @pl.when(e == 0)
    def _():
        out_ref[...] = jnp.zeros_like(out_ref)

    w_all = w_scr[...]
    onehot = jax.lax.broadcasted_iota(jnp.int32, w_all.shape, 1) == e
    w_e = jnp.sum(jnp.where(onehot, w_all, 0.0), axis=1, keepdims=True)
    y = jax.lax.dot_general(
        x_ref[...], ew_ref[0],
        (((1,), (1,)), ((), ())),
        preferred_element_type=jnp.float32,
    ) + eb_ref[0]
    out_ref[...] += y * w_e


@functools.partial(jax.jit, static_argnames=("interpret",))
def kernel(x, router_W, router_b, expert_weight, expert_bias, interpret=False):
    B, D = x.shape
    E, O, _ = expert_weight.shape
    ON = O // NS
    rb2 = router_b.reshape(1, E)
    eb3 = expert_bias.reshape(E, 1, O)
    grid = (B // BM, NS, E)
    out = pl.pallas_call(
        _moe_body,
        grid=grid,
        in_specs=[
            pl.BlockSpec((BM, D), lambda t, n, e: (t, 0)),
            pl.BlockSpec((E, D), lambda t, n, e: (0, 0)),
            pl.BlockSpec((1, E), lambda t, n, e: (0, 0)),
            pl.BlockSpec((1, ON, D), lambda t, n, e: (e, n, 0)),
            pl.BlockSpec((1, 1, ON), lambda t, n, e: (e, 0, n)),
        ],
        out_specs=pl.BlockSpec((BM, ON), lambda t, n, e: (t, n)),
        out_shape=jax.ShapeDtypeStruct((B, O), jnp.float32),
        scratch_shapes=[pltpu.VMEM((BM, NUM_EXPERTS), jnp.float32)],
        interpret=interpret,
    )(x, router_W, rb2, expert_weight, eb3)
    return out


# speedup vs baseline: 1.6446x; 1.0013x over previous
import functools

import jax
import jax.numpy as jnp
from jax.experimental import pallas as pl
from jax.experimental.pallas import tpu as pltpu

NUM_EXPERTS = 8
TOP_K = 2
BM = 2048
NS = 4


def _topk2_weights(logits):
    E = logits.shape[1]
    iota = jax.lax.broadcasted_iota(jnp.int32, logits.shape, 1)
    m1 = jnp.max(logits, axis=1, keepdims=True)
    eq1 = logits == m1
    idx1 = jnp.min(jnp.where(eq1, iota, E), axis=1, keepdims=True)
    first = iota == idx1
    masked = jnp.where(first, -jnp.inf, logits)
    m2 = jnp.max(masked, axis=1, keepdims=True)
    eq2 = masked == m2
    idx2 = jnp.min(jnp.where(eq2, iota, E), axis=1, keepdims=True)
    second = iota == idx2
    p = jnp.exp(m2 - m1)
    denom = 1.0 + p
    w = (first.astype(jnp.float32) + second.astype(jnp.float32) * p) / denom
    return w


def _moe_body(x_ref, rw_ref, rb_ref, ew_ref, eb_ref, out_ref, w_scr):
    n = pl.program_id(1)
    e = pl.program_id(2)

    @pl.when((n == 0) & (e == 0))
    def _():
        logits = jax.lax.dot_general(
            x_ref[...], rw_ref[...],
            (((1,), (1,)), ((), ())),
            preferred_element_type=jnp.float32,
        ) + rb_ref[...]
        w_scr[...] = _topk2_weights(logits)

    @pl.when(e == 0)
    def _():
        out_ref[...] = jnp.zeros_like(out_ref)

    w_all = w_scr[...]
    onehot = jax.lax.broadcasted_iota(jnp.int32, w_all.shape, 1) == e
    w_e = jnp.sum(jnp.where(onehot, w_all, 0.0), axis=1, keepdims=True)
    y = jax.lax.dot_general(
        x_ref[...], ew_ref[0],
        (((1,), (1,)), ((), ())),
        preferred_element_type=jnp.float32,
    ) + eb_ref[0]
    out_ref[...] += y * w_e


@jax.jit
def kernel(x, router_W, router_b, expert_weight, expert_bias):
    B, D = x.shape
    E, O, _ = expert_weight.shape
    ON = O // NS
    rb2 = router_b.reshape(1, E)
    eb3 = expert_bias.reshape(E, 1, O)
    grid = (B // BM, NS, E)
    out = pl.pallas_call(
        _moe_body,
        grid=grid,
        in_specs=[
            pl.BlockSpec((BM, D), lambda t, n, e: (t, 0)),
            pl.BlockSpec((E, D), lambda t, n, e: (0, 0)),
            pl.BlockSpec((1, E), lambda t, n, e: (0, 0)),
            pl.BlockSpec((1, ON, D), lambda t, n, e: (e, n, 0)),
            pl.BlockSpec((1, 1, ON), lambda t, n, e: (e, 0, n)),
        ],
        out_specs=pl.BlockSpec((BM, ON), lambda t, n, e: (t, n)),
        out_shape=jax.ShapeDtypeStruct((B, O), jnp.float32),
        scratch_shapes=[pltpu.VMEM((BM, NUM_EXPERTS), jnp.float32)],
    )(x, router_W, rb2, expert_weight, eb3)
    return out
